# baseline (speedup 1.0000x reference)
import functools

import jax
import jax.numpy as jnp
from jax import lax
from jax.experimental import pallas as pl
from jax.experimental.pallas import tpu as pltpu
from jax.experimental.pallas import tpu_sc as plsc

_LANES = 16


@functools.lru_cache(maxsize=None)
def _make_sc_add(B: int, S: int, D: int):
    info = plsc.get_sparse_core_info()
    NC, NS = info.num_cores, info.num_subcores
    NW = NC * NS

    rows_per_w = S // NW
    T_ROWS = 32
    HALF = T_ROWS // 2
    n_tiles = rows_per_w // T_ROWS
    assert S % NW == 0 and rows_per_w % T_ROWS == 0 and D % _LANES == 0

    mesh = plsc.VectorSubcoreMesh(core_axis_name="c", subcore_axis_name="s")

    @functools.partial(
        pl.kernel,
        mesh=mesh,
        out_type=jax.ShapeDtypeStruct((B * S, D), jnp.float32),
        scratch_types=(
            [pltpu.VMEM((T_ROWS, D), jnp.float32)]
            + [pltpu.VMEM((T_ROWS, D), jnp.float32)] * 2
            + [pltpu.SemaphoreType.DMA] * 4
            + [pltpu.SemaphoreType.DMA] * 4
            + [pltpu.SemaphoreType.DMA]
        ),
    )
    def k(x_hbm, pe_hbm, out_hbm, pebuf, xb0, xb1, *sems):
        xb = (xb0, xb1)
        ls = (sems[0:2], sems[2:4])
        ss = (sems[4:6], sems[6:8])
        pes = sems[8]
        wid = lax.axis_index("s") * NC + lax.axis_index("c")
        w_row = wid * rows_per_w

        steps = [(t, b) for t in range(n_tiles) for b in range(B)]
        n = len(steps)

        def x_rows(i, h):
            t, b = steps[i]
            return pl.ds(b * S + w_row + t * T_ROWS + h * HALF, HALF)

        def start_load(i):
            p = i % 2
            return [pltpu.async_copy(
                x_hbm.at[x_rows(i, h)], xb[p].at[pl.ds(h * HALF, HALF)],
                ls[p][h]) for h in range(2)]

        h_store = [None] * n
        h_pe = pltpu.async_copy(pe_hbm.at[pl.ds(w_row, T_ROWS)], pebuf, pes)
        h_load = start_load(0)
        for i in range(n):
            t, b = steps[i]
            p = i % 2
            if i + 1 < n:
                if i - 1 >= 0:
                    for hs in h_store[i - 1]:
                        hs.wait()
                nxt = start_load(i + 1)
            if b == 0:
                h_pe.wait()
            xbp = xb[p]
            h_store[i] = []
            for h in range(2):
                h_load[h].wait()

                @plsc.parallel_loop(h * HALF, (h + 1) * HALF, unroll=1)
                def add_body(r):
                    @plsc.parallel_loop(0, 2, unroll=1)
                    def add_cols(q):
                        for c in range(D // _LANES // 2):
                            sl = pl.ds(q * (D // 2) + c * _LANES, _LANES)
                            xbp[r, sl] = xbp[r, sl] + pebuf[r, sl]

                h_store[i].append(pltpu.async_copy(
                    xbp.at[pl.ds(h * HALF, HALF)],
                    out_hbm.at[x_rows(i, h)], ss[p][h]))
            if i + 1 < n:
                h_load = nxt
            if b == B - 1 and t + 1 < n_tiles:
                h_pe = pltpu.async_copy(
                    pe_hbm.at[pl.ds(w_row + (t + 1) * T_ROWS, T_ROWS)],
                    pebuf, pes)
        for i in (n - 2, n - 1):
            for hs in h_store[i]:
                hs.wait()

    return k


def kernel(x, pe_emb):
    B, S, D = x.shape
    k = _make_sc_add(B, S, D)
    out = k(x.reshape(B * S, D), pe_emb)
    return out.reshape(B, S, D)


# speedup vs baseline: 2.1045x; 2.1045x over previous
import functools

import jax
import jax.numpy as jnp
from jax import lax
from jax.experimental import pallas as pl
from jax.experimental.pallas import tpu as pltpu
from jax.experimental.pallas import tpu_sc as plsc

_LANES = 16


@functools.lru_cache(maxsize=None)
def _make_sc_add(B: int, S: int, D: int):
    info = plsc.get_sparse_core_info()
    NC, NS = info.num_cores, info.num_subcores
    NW = NC * NS

    rows_per_w = S // NW
    T_ROWS = 32
    HALF = T_ROWS // 2
    n_tiles = rows_per_w // T_ROWS
    assert S % NW == 0 and rows_per_w % T_ROWS == 0 and D % _LANES == 0
    assert n_tiles >= 3

    mesh = plsc.VectorSubcoreMesh(core_axis_name="c", subcore_axis_name="s")

    @functools.partial(
        pl.kernel,
        mesh=mesh,
        out_type=jax.ShapeDtypeStruct((B * S, D), jnp.float32),
        scratch_types=(
            [pltpu.VMEM((T_ROWS, D), jnp.float32)]
            + [pltpu.VMEM((T_ROWS, D), jnp.float32)] * 2
            + [pltpu.SemaphoreType.DMA] * 4
            + [pltpu.SemaphoreType.DMA] * 4
            + [pltpu.SemaphoreType.DMA]
        ),
    )
    def k(x_hbm, pe_hbm, out_hbm, pebuf, xb0, xb1, *sems):
        xb = (xb0, xb1)
        ls = (sems[0:2], sems[2:4])
        ss = (sems[4:6], sems[6:8])
        pes = sems[8]
        wid = lax.axis_index("s") * NC + lax.axis_index("c")
        w_row = wid * rows_per_w

        def x_rows(t, b, h):
            return pl.ds(b * S + w_row + t * T_ROWS + h * HALF, HALF)

        def half(buf, h):
            return buf.at[pl.ds(h * HALF, HALF)]

        def issue_load(t, b, p):
            for h in range(2):
                pltpu.async_copy(x_hbm.at[x_rows(t, b, h)], half(xb[p], h),
                                 ls[p][h])

        def wait_load(p, h):
            pltpu.make_async_copy(
                x_hbm.at[pl.ds(0, HALF)], half(xb[p], h), ls[p][h]).wait()

        def drain_store(p):
            for h in range(2):
                pltpu.make_async_copy(
                    half(xb[p], h), out_hbm.at[pl.ds(0, HALF)],
                    ss[p][h]).wait()

        def pe_start(t):
            pltpu.async_copy(pe_hbm.at[pl.ds(w_row + t * T_ROWS, T_ROWS)],
                             pebuf, pes)

        def pe_wait():
            pltpu.make_async_copy(
                pe_hbm.at[pl.ds(0, T_ROWS)], pebuf, pes).wait()

        def add_store(t, b, p):
            xbp = xb[p]
            for h in range(2):
                wait_load(p, h)

                @plsc.parallel_loop(h * HALF, (h + 1) * HALF, unroll=1)
                def add_body(r):
                    for c in range(D // _LANES):
                        sl = pl.ds(c * _LANES, _LANES)
                        xbp[r, sl] = xbp[r, sl] + pebuf[r, sl]

                pltpu.async_copy(half(xbp, h), out_hbm.at[x_rows(t, b, h)],
                                 ss[p][h])


        pe_start(0)
        issue_load(0, 0, 0)
        for b in range(B):
            p = b % 2
            if b > 0:
                drain_store(p ^ 1)
            if b + 1 < B:
                issue_load(0, b + 1, p ^ 1)
            else:
                issue_load(1, 0, p ^ 1)
            if b == 0:
                pe_wait()
            add_store(0, b, p)
        pe_start(1)

        def tile_body(t, carry):
            for b in range(B):
                p = b % 2
                drain_store(p ^ 1)
                if b + 1 < B:
                    issue_load(t, b + 1, p ^ 1)
                else:
                    issue_load(t + 1, 0, p ^ 1)
                if b == 0:
                    pe_wait()
                add_store(t, b, p)
            pe_start(t + 1)
            return carry

        lax.fori_loop(1, n_tiles - 1, tile_body, 0)

        tl = n_tiles - 1
        for b in range(B):
            p = b % 2
            drain_store(p ^ 1)
            if b + 1 < B:
                issue_load(tl, b + 1, p ^ 1)
            if b == 0:
                pe_wait()
            add_store(tl, b, p)
        drain_store((B - 1) % 2)

    return k


def kernel(x, pe_emb):
    B, S, D = x.shape
    k = _make_sc_add(B, S, D)
    out = k(x.reshape(B * S, D), pe_emb)
    return out.reshape(B, S, D)
